# Initial kernel scaffold; baseline (speedup 1.0000x reference)
import jax
import jax.numpy as jnp
from jax.experimental import pallas as pl


def kernel(indices, weight):
    raise NotImplementedError("write your pallas kernel here")


# speedup vs baseline: 1.0218x; 1.0218x over previous
import functools

import jax
import jax.numpy as jnp
from jax import lax
from jax.experimental import pallas as pl
from jax.experimental.pallas import tpu as pltpu
from jax.experimental.pallas import tpu_sc as plsc

BATCH = 16384
HIST = 50
DIM = 32
B = BATCH * HIST
NC = 2
NS = 16
NW = NC * NS
CHUNK = 128
PER_W = B // NW
NCH = PER_W // CHUNK


def _build_gather():
    mesh = plsc.VectorSubcoreMesh(core_axis_name="c", subcore_axis_name="s")

    @functools.partial(
        pl.kernel,
        mesh=mesh,
        out_type=jax.ShapeDtypeStruct((B, DIM), jnp.float32),
        compiler_params=pltpu.CompilerParams(use_tc_tiling_on_sc=False),
        scratch_types=[
            pltpu.VMEM((NCH, CHUNK), jnp.int32),
            pltpu.VMEM((CHUNK, DIM), jnp.float32),
            pltpu.SemaphoreType.DMA,
        ],
    )
    def gather_kernel(idx_hbm, table_hbm, out_hbm, idx_v, rows_v, sem):
        wid = lax.axis_index("s") * NC + lax.axis_index("c")
        base = wid * PER_W
        pltpu.sync_copy(idx_hbm.at[wid], idx_v)

        def body(j, carry):
            pltpu.async_copy(table_hbm.at[idx_v.at[j]], rows_v, sem).wait()
            start = pl.multiple_of(base + j * CHUNK, CHUNK)
            pltpu.sync_copy(rows_v, out_hbm.at[pl.ds(start, CHUNK)])
            return carry

        lax.fori_loop(0, NCH, body, 0)

    return gather_kernel


_gather = _build_gather()


def kernel(indices, weight):
    idx = indices.reshape(NW, NCH, CHUNK).astype(jnp.int32)
    out = _gather(idx, weight)
    return out.reshape(BATCH, HIST, DIM)


# speedup vs baseline: 1.1125x; 1.0887x over previous
import functools

import jax
import jax.numpy as jnp
from jax import lax
from jax.experimental import pallas as pl
from jax.experimental.pallas import tpu as pltpu
from jax.experimental.pallas import tpu_sc as plsc

BATCH = 16384
HIST = 50
DIM = 32
B = BATCH * HIST
NC = 2
NS = 16
NW = NC * NS
CHUNK = 128
PER_W = B // NW
NCH = PER_W // CHUNK


NBUF = 8
GDEPTH = 6
NGRP = NCH // NBUF


def _build_gather():
    mesh = plsc.VectorSubcoreMesh(core_axis_name="c", subcore_axis_name="s")

    @functools.partial(
        pl.kernel,
        mesh=mesh,
        out_type=jax.ShapeDtypeStruct((B, DIM), jnp.float32),
        compiler_params=pltpu.CompilerParams(use_tc_tiling_on_sc=False),
        scratch_types=[
            pltpu.VMEM((NCH, CHUNK), jnp.int32),
            pltpu.VMEM((NBUF, CHUNK, DIM), jnp.float32),
            pltpu.SemaphoreType.DMA((NBUF,)),
            pltpu.SemaphoreType.DMA((NBUF,)),
        ],
    )
    def gather_kernel(idx_hbm, table_hbm, out_hbm, idx_v, rows_v, gsem, ssem):
        wid = lax.axis_index("s") * NC + lax.axis_index("c")
        base = wid * PER_W
        pltpu.sync_copy(idx_hbm.at[wid], idx_v)

        def start_gather(j, b):
            pltpu.async_copy(table_hbm.at[idx_v.at[j]], rows_v.at[b], gsem.at[b])

        def wait_gather(b):
            pltpu.make_async_copy(
                table_hbm.at[idx_v.at[0]], rows_v.at[b], gsem.at[b]
            ).wait()

        def out_slice(j):
            start = pl.multiple_of(base + j * CHUNK, CHUNK)
            return out_hbm.at[pl.ds(start, CHUNK)]

        def start_store(j, b):
            pltpu.async_copy(rows_v.at[b], out_slice(j), ssem.at[b])

        def wait_store(j, b):
            pltpu.make_async_copy(rows_v.at[b], out_slice(j), ssem.at[b]).wait()

        for b in range(GDEPTH):
            start_gather(b, b)

        def body(g, carry):
            for b in range(NBUF):
                j = g * NBUF + b
                wait_gather(b)
                start_store(j, b)
                jj = j + GDEPTH
                bb = (b + GDEPTH) % NBUF

                @pl.when(jj < NCH)
                def _():
                    @pl.when(jj >= NBUF)
                    def _():
                        wait_store(jj - NBUF, bb)

                    start_gather(jj, bb)

            return carry

        lax.fori_loop(0, NGRP, body, 0)

        for b in range(NBUF):
            wait_store(NCH - NBUF + b, b)

    return gather_kernel


_gather = _build_gather()


def kernel(indices, weight):
    idx = indices.reshape(NW, NCH, CHUNK).astype(jnp.int32)
    out = _gather(idx, weight)
    return out.reshape(BATCH, HIST, DIM)


# speedup vs baseline: 1.4209x; 1.2772x over previous
import functools

import jax
import jax.numpy as jnp
from jax import lax
from jax.experimental import pallas as pl
from jax.experimental.pallas import tpu as pltpu
from jax.experimental.pallas import tpu_sc as plsc

BATCH = 16384
HIST = 50
DIM = 32
NC = 2
NS = 16
NW = NC * NS
CHUNK = 128
BPW = BATCH // NW
NSUB = BPW // CHUNK
NCH = HIST * NSUB
NBUF = 8
GDEPTH = 6
NGRP = NCH // NBUF


def _build_gather():
    mesh = plsc.VectorSubcoreMesh(core_axis_name="c", subcore_axis_name="s")

    @functools.partial(
        pl.kernel,
        mesh=mesh,
        out_type=jax.ShapeDtypeStruct((HIST, DIM, BATCH), jnp.float32),
        compiler_params=pltpu.CompilerParams(
            use_tc_tiling_on_sc=False, needs_layout_passes=False
        ),
        scratch_types=[
            pltpu.VMEM((HIST, NSUB, CHUNK), jnp.int32),
            pltpu.VMEM((NBUF, CHUNK, DIM), jnp.float32),
            pltpu.VMEM((NBUF, DIM, CHUNK), jnp.float32),
            pltpu.SemaphoreType.DMA((NBUF,)),
            pltpu.SemaphoreType.DMA((NBUF,)),
        ],
    )
    def gather_kernel(idx_hbm, table_hbm, out_hbm, idx_v, g_v, t_v, gsem, ssem):
        wid = lax.axis_index("s") * NC + lax.axis_index("c")
        b0 = wid * BPW
        pltpu.sync_copy(idx_hbm.at[:, pl.ds(wid * NSUB, NSUB), :], idx_v)

        iota = lax.iota(jnp.int32, 16)

        def start_gather(k, b):
            h = k // NSUB
            sub = k % NSUB
            pltpu.async_copy(table_hbm.at[idx_v.at[h, sub]], g_v.at[b], gsem.at[b])

        def wait_gather(b):
            pltpu.make_async_copy(
                table_hbm.at[idx_v.at[0, 0]], g_v.at[b], gsem.at[b]
            ).wait()

        def out_slice(k):
            h = k // NSUB
            sub = k % NSUB
            start = pl.multiple_of(b0 + sub * CHUNK, CHUNK)
            return out_hbm.at[h, :, pl.ds(start, CHUNK)]

        def start_store(k, b):
            pltpu.async_copy(t_v.at[b], out_slice(k), ssem.at[b])

        def wait_store(k, b):
            pltpu.make_async_copy(t_v.at[b], out_slice(k), ssem.at[b]).wait()

        def transpose_chunk(b):
            for d in range(DIM):
                col = jnp.full((16,), d, jnp.int32)
                for r0 in range(0, CHUNK, 16):
                    v = plsc.load_gather(g_v.at[b], [iota + r0, col])
                    t_v[b, d, pl.ds(r0, 16)] = v

        for b in range(GDEPTH):
            start_gather(b, b)

        def body(g, carry):
            for b in range(NBUF):
                k = g * NBUF + b
                wait_gather(b)

                @pl.when(k >= NBUF)
                def _():
                    wait_store(k - NBUF, b)

                transpose_chunk(b)
                start_store(k, b)
                kk = k + GDEPTH

                @pl.when(kk < NCH)
                def _():
                    start_gather(kk, (b + GDEPTH) % NBUF)

            return carry

        lax.fori_loop(0, NGRP, body, 0)

        for b in range(NBUF):
            wait_store(NCH - NBUF + b, b)

    return gather_kernel


_gather = _build_gather()


def kernel(indices, weight):
    idx3 = indices.T.reshape(HIST, BATCH // CHUNK, CHUNK).astype(jnp.int32)
    out3 = _gather(idx3, weight)
    return out3.transpose(2, 0, 1)


# speedup vs baseline: 1.9113x; 1.3452x over previous
import functools

import jax
import jax.numpy as jnp
from jax import lax
from jax.experimental import pallas as pl
from jax.experimental.pallas import tpu as pltpu
from jax.experimental.pallas import tpu_sc as plsc

BATCH = 16384
HIST = 50
DIM = 32
NC = 2
NS = 16
NW = NC * NS
CHUNK = 128
BPW = BATCH // NW
NSUB = BPW // CHUNK
NCH = HIST * NSUB
NBUF = 8
GDEPTH = 6
NGRP = NCH // NBUF


def _build_gather():
    mesh = plsc.VectorSubcoreMesh(core_axis_name="c", subcore_axis_name="s")

    @functools.partial(
        pl.kernel,
        mesh=mesh,
        out_type=jax.ShapeDtypeStruct((HIST, DIM, BATCH), jnp.float32),
        compiler_params=pltpu.CompilerParams(
            use_tc_tiling_on_sc=False, needs_layout_passes=False
        ),
        scratch_types=[
            pltpu.VMEM((HIST, NSUB, CHUNK), jnp.int32),
            pltpu.VMEM((NBUF, CHUNK, DIM), jnp.float32),
            pltpu.VMEM((NBUF, DIM, CHUNK), jnp.float32),
            pltpu.SemaphoreType.DMA((NBUF,)),
            pltpu.SemaphoreType.DMA((NBUF,)),
        ],
    )
    def gather_kernel(idx_hbm, table_hbm, out_hbm, idx_v, g_v, t_v, gsem, ssem):
        wid = lax.axis_index("s") * NC + lax.axis_index("c")
        b0 = wid * BPW
        pltpu.sync_copy(idx_hbm.at[:, pl.ds(wid * NSUB, NSUB), :], idx_v)

        iota = lax.iota(jnp.int32, 16)

        def start_gather(k, b):
            h = k // NSUB
            sub = k % NSUB
            pltpu.async_copy(table_hbm.at[idx_v.at[h, sub]], g_v.at[b], gsem.at[b])

        def wait_gather(b):
            pltpu.make_async_copy(
                table_hbm.at[idx_v.at[0, 0]], g_v.at[b], gsem.at[b]
            ).wait()

        def out_slice(k):
            h = k // NSUB
            sub = k % NSUB
            start = pl.multiple_of(b0 + sub * CHUNK, CHUNK)
            return out_hbm.at[h, :, pl.ds(start, CHUNK)]

        def start_store(k, b):
            pltpu.async_copy(t_v.at[b], out_slice(k), ssem.at[b])

        def wait_store(k, b):
            pltpu.make_async_copy(t_v.at[b], out_slice(k), ssem.at[b]).wait()

        def transpose_chunk(b):
            @plsc.parallel_loop(0, DIM, 1, unroll=8)
            def _(d):
                col = jnp.full((16,), d, jnp.int32)
                for r0 in range(0, CHUNK, 16):
                    v = plsc.load_gather(g_v.at[b], [iota + r0, col])
                    t_v[b, d, pl.ds(r0, 16)] = v

        for b in range(GDEPTH):
            start_gather(b, b)

        def body(g, carry):
            for b in range(NBUF):
                k = g * NBUF + b
                wait_gather(b)

                @pl.when(k >= NBUF)
                def _():
                    wait_store(k - NBUF, b)

                transpose_chunk(b)
                start_store(k, b)
                kk = k + GDEPTH

                @pl.when(kk < NCH)
                def _():
                    start_gather(kk, (b + GDEPTH) % NBUF)

            return carry

        lax.fori_loop(0, NGRP, body, 0)

        for b in range(NBUF):
            wait_store(NCH - NBUF + b, b)

    return gather_kernel


_gather = _build_gather()


def kernel(indices, weight):
    idx3 = indices.T.reshape(HIST, BATCH // CHUNK, CHUNK).astype(jnp.int32)
    out3 = _gather(idx3, weight)
    return out3.transpose(2, 0, 1)
